# baseline (speedup 1.0000x reference)
import jax
import jax.numpy as jnp
from jax.experimental import pallas as pl


def _st_embed_kernel(wv_ref, wt_ref, out_ref):
    wv = wv_ref[...]
    wt = wt_ref[...]
    out_ref[...] = wv[None, :, None, :] + wt[None, None, :, :]


def kernel(x, W_veh, W_time, W_pos):
    B, N, T, F = x.shape
    D = W_veh.shape[1]
    return pl.pallas_call(
        _st_embed_kernel,
        grid=(B,),
        in_specs=[
            pl.BlockSpec((N, D), lambda i: (0, 0)),
            pl.BlockSpec((T, D), lambda i: (0, 0)),
        ],
        out_specs=pl.BlockSpec((1, N, T, D), lambda i: (i, 0, 0, 0)),
        out_shape=jax.ShapeDtypeStruct((B, N, T, D), W_veh.dtype),
    )(W_veh[:N], W_time[:T])
